# baseline (speedup 1.0000x reference)
import functools

import jax
import jax.numpy as jnp
from jax import lax
from jax.experimental import pallas as pl
from jax.experimental.pallas import tpu as pltpu
from jax.experimental.pallas import tpu_sc as plsc

_INPUT_DIM = 39
_N_CAT = 26
_VOCAB = 100000
_EMB = 16
_BATCH = 16384
_N_CONT = _INPUT_DIM - _N_CAT
_OUT_DIM = _N_CONT + _N_CAT * _EMB

_NC = 2
_NS = 16
_NW = _NC * _NS

_B_PER_W = _BATCH // _NW
_CHUNK = 128
_N_CHUNKS = _B_PER_W // _CHUNK

_L = 16


@functools.partial(
    pl.kernel,
    mesh=plsc.VectorSubcoreMesh(core_axis_name="c", subcore_axis_name="s"),
    out_type=jax.ShapeDtypeStruct((_OUT_DIM, _BATCH), jnp.float32),
    scratch_types=[
        pltpu.VMEM((_INPUT_DIM, _CHUNK), jnp.int32),
        pltpu.VMEM((_OUT_DIM, _CHUNK), jnp.float32),
        pltpu.SemaphoreType.DMA,
    ],
    compiler_params=pltpu.CompilerParams(
        use_tc_tiling_on_sc=False, needs_layout_passes=False
    ),
)
def _sc_embed(xt_hbm, tabt_hbm, out_hbm, xt_v, out_v, sem):
    wid = lax.axis_index("s") * _NC + lax.axis_index("c")
    w0 = wid * _B_PER_W

    def chunk_body(c, carry):
        b0 = w0 + c * _CHUNK
        pltpu.sync_copy(xt_hbm.at[:, pl.ds(b0, _CHUNK)], xt_v)

        def feat_body(j, carry2):
            for e in range(_EMB):
                pltpu.async_copy(
                    tabt_hbm.at[j, e].at[xt_v.at[_N_CONT + j]],
                    out_v.at[_N_CONT + j * _EMB + e],
                    sem,
                )
            return carry2

        lax.fori_loop(0, _N_CAT, feat_body, 0)

        for col in range(_N_CONT):
            for g in range(_CHUNK // _L):
                vals = xt_v[col, pl.ds(g * _L, _L)]
                out_v[col, pl.ds(g * _L, _L)] = vals.astype(jnp.float32)

        def drain_body(k, carry2):
            pltpu.make_async_copy(
                tabt_hbm.at[0, 0, pl.ds(0, _CHUNK)],
                out_v.at[_N_CONT],
                sem,
            ).wait()
            return carry2

        lax.fori_loop(0, _N_CAT * _EMB, drain_body, 0)

        pltpu.sync_copy(out_v, out_hbm.at[:, pl.ds(b0, _CHUNK)])
        return carry

    lax.fori_loop(0, _N_CHUNKS, chunk_body, 0)


def kernel(x, tables):
    out_t = _sc_embed(x.T, tables.transpose(0, 2, 1))
    return out_t.T


# speedup vs baseline: 1.0116x; 1.0116x over previous
import functools

import jax
import jax.numpy as jnp
from jax import lax
from jax.experimental import pallas as pl
from jax.experimental.pallas import tpu as pltpu
from jax.experimental.pallas import tpu_sc as plsc

_INPUT_DIM = 39
_N_CAT = 26
_VOCAB = 100000
_EMB = 16
_BATCH = 16384
_N_CONT = _INPUT_DIM - _N_CAT
_OUT_DIM = _N_CONT + _N_CAT * _EMB

_NC = 2
_NS = 16
_NW = _NC * _NS

_B_PER_W = _BATCH // _NW
_CHUNK = 128
_N_CHUNKS = _B_PER_W // _CHUNK

_L = 16


@functools.partial(
    pl.kernel,
    mesh=plsc.VectorSubcoreMesh(core_axis_name="c", subcore_axis_name="s"),
    out_type=jax.ShapeDtypeStruct((_OUT_DIM, _BATCH), jnp.float32),
    scratch_types=[
        pltpu.VMEM((2, _INPUT_DIM, _CHUNK), jnp.int32),
        pltpu.VMEM((2, _OUT_DIM, _CHUNK), jnp.float32),
        pltpu.SemaphoreType.DMA,
        pltpu.SemaphoreType.DMA,
    ],
    compiler_params=pltpu.CompilerParams(
        use_tc_tiling_on_sc=False, needs_layout_passes=False
    ),
)
def _sc_embed(xt_hbm, tabt_hbm, out_hbm, xt_v, out_v, sem0, sem1):
    wid = lax.axis_index("s") * _NC + lax.axis_index("c")
    w0 = wid * _B_PER_W
    sems = (sem0, sem1)

    def fire_chunk(c):
        buf = c % 2
        b0 = w0 + c * _CHUNK
        pltpu.sync_copy(xt_hbm.at[:, pl.ds(b0, _CHUNK)], xt_v.at[buf])

        def feat_body(j, carry):
            for e in range(_EMB):
                pltpu.async_copy(
                    tabt_hbm.at[j, e].at[xt_v.at[buf, _N_CONT + j]],
                    out_v.at[buf, _N_CONT + j * _EMB + e],
                    sems[buf],
                )
            return carry

        lax.fori_loop(0, _N_CAT, feat_body, 0)

        for col in range(_N_CONT):
            for g in range(_CHUNK // _L):
                vals = xt_v[buf, col, pl.ds(g * _L, _L)]
                out_v[buf, col, pl.ds(g * _L, _L)] = vals.astype(jnp.float32)

    def finish_chunk(c):
        buf = c % 2
        b0 = w0 + c * _CHUNK

        def drain_body(k, carry):
            pltpu.make_async_copy(
                tabt_hbm.at[0, 0, pl.ds(0, _CHUNK)],
                out_v.at[buf, _N_CONT],
                sems[buf],
            ).wait()
            return carry

        lax.fori_loop(0, _N_CAT * _EMB, drain_body, 0)
        pltpu.sync_copy(out_v.at[buf], out_hbm.at[:, pl.ds(b0, _CHUNK)])

    fire_chunk(0)
    for c in range(1, _N_CHUNKS):
        fire_chunk(c)
        finish_chunk(c - 1)
    finish_chunk(_N_CHUNKS - 1)


def kernel(x, tables):
    out_t = _sc_embed(x.T, tables.transpose(0, 2, 1))
    return out_t.T
